# baseline (speedup 1.0000x reference)
import functools

import jax
import jax.numpy as jnp
from jax import lax
from jax.experimental import pallas as pl
from jax.experimental.pallas import tpu as pltpu
from jax.experimental.pallas import tpu_sc as plsc

_NC = 2
_NS = 16
_NW = _NC * _NS
_L = 16


def _sc_body(x_hbm, tab_hbm, o_hbm, tabv, buf, *, per_w, ch, b):
    c = lax.axis_index("c")
    s = lax.axis_index("s")
    wid = s * _NC + c
    row0 = wid * per_w
    nch = per_w // ch
    nvec = b // _L
    pltpu.sync_copy(tab_hbm.at[pl.ds(row0, per_w)], tabv)

    def chunk(j, carry):
        base = row0 + j * ch
        pltpu.sync_copy(x_hbm.at[pl.ds(base, ch)], buf)
        for srow in range(ch):
            tsp = jnp.reshape(tabv[pl.ds(j * ch + srow, 1), :], (_L,))

            def vec(k, carry2):
                for u in range(8):
                    off = (k * 8 + u) * _L
                    buf[srow, pl.ds(off, _L)] = buf[srow, pl.ds(off, _L)] + tsp
                return carry2

            lax.fori_loop(0, nvec // 8, vec, 0)
        pltpu.sync_copy(buf, o_hbm.at[pl.ds(base, ch)])
        return carry

    lax.fori_loop(0, nch, chunk, 0)


def sc_add(x, tab16):
    R, B = x.shape
    per_w = R // _NW
    ch = 8
    mesh = plsc.VectorSubcoreMesh(core_axis_name="c", subcore_axis_name="s")
    return pl.kernel(
        functools.partial(_sc_body, per_w=per_w, ch=ch, b=B),
        out_type=jax.ShapeDtypeStruct((R, B), jnp.float32),
        mesh=mesh,
        scratch_types=[
            pltpu.VMEM((per_w, _L), jnp.float32),
            pltpu.VMEM((ch, B), jnp.float32),
        ],
    )(x, tab16)


def kernel(inputs, lookup_table):
    B, T, U = inputs.shape
    scale = float(U) ** 0.5
    x = jnp.transpose(inputs, (1, 2, 0)).reshape(T * U, B)
    tab16 = jnp.broadcast_to(
        (lookup_table.reshape(T * U) * scale)[:, None], (T * U, _L)
    )
    out = sc_add(x, tab16)
    return jnp.transpose(out.reshape(T, U, B), (2, 0, 1))
